# baseline (speedup 1.0000x reference)
import jax
import jax.numpy as jnp
from jax import lax
from jax.experimental import pallas as pl
from jax.experimental.pallas import tpu as pltpu
from jax.experimental.pallas import tpu_sc as plsc

E = 128
B_SEG = 1024
N_NODES = 100000
NC, NS, L = 2, 16, 16
SEG_PER_SC = B_SEG // NC
SEG_PER_TILE = SEG_PER_SC // NS
CHUNK = 3200
CPAD = CHUNK + 16
BINS = SEG_PER_SC * E
DUMMY = BINS
HIST_PAD = BINS + 128
STRIPE = SEG_PER_TILE * E
SEARCH_STEPS = 17


def _sc_hist_body(z_hbm, batch_hbm, out_hbm, zb_v, bb_v, keys_v, ones_v,
                  zero_v, probe_v, probe_s, hist_sh, probe_sh, sem):
    cid = lax.axis_index("c")
    sid = lax.axis_index("s")
    seg0_sc = cid * SEG_PER_SC
    seg0 = seg0_sc + sid * SEG_PER_TILE

    def _ones(i, _):
        ones_v[pl.ds(pl.multiple_of(i * L, L), L)] = (
            jnp.full((L,), 1.0, jnp.float32))
        return 0
    lax.fori_loop(0, CPAD // L, _ones, 0)

    def _zero(i, _):
        zero_v[pl.ds(pl.multiple_of(i * L, L), L)] = (
            jnp.zeros((L,), jnp.float32))
        return 0
    lax.fori_loop(0, STRIPE // L, _zero, 0)
    pltpu.sync_copy(zero_v,
                    hist_sh.at[pl.ds(pl.multiple_of(sid * STRIPE, 8), STRIPE)])

    iota = lax.broadcasted_iota(jnp.int32, (L,), 0)
    tgt = jnp.where(iota < 1, seg0, seg0 + SEG_PER_TILE)

    def _bstep(_, st):
        lo_v, hi_v = st
        mid = jnp.right_shift(lo_v + hi_v, 1)
        cp = pltpu.async_copy(batch_hbm.at[jnp.minimum(mid, N_NODES - 1)],
                              probe_v, sem)
        cp.wait()
        v = probe_v[...]
        c = jnp.where(lo_v < hi_v, jnp.where(v < tgt, 1, 2), 0)
        return (jnp.where(c == 1, mid + 1, lo_v),
                jnp.where(c == 2, mid, hi_v))

    zv = jnp.zeros((L,), jnp.int32)
    lo_v, _ = lax.fori_loop(0, SEARCH_STEPS, _bstep, (zv, zv + N_NODES))
    probe_v[...] = lo_v
    pslot = pl.multiple_of(sid * L, 8)
    pltpu.sync_copy(probe_v, probe_sh.at[pl.ds(pslot, L)])
    pltpu.sync_copy(probe_sh.at[pl.ds(pslot, L)], probe_s)
    beg = probe_s[0]
    end = probe_s[1]

    key_base = seg0_sc * E

    def _chunk(k, _):
        lo = beg + k * CHUNK
        hi = jnp.minimum(lo + CHUNK, end)
        a0 = pl.multiple_of(jnp.minimum(lo & ~7, N_NODES - CPAD), 8)
        cz = pltpu.async_copy(z_hbm.at[pl.ds(a0, CPAD)], zb_v, sem)
        cb = pltpu.async_copy(batch_hbm.at[pl.ds(a0, CPAD)], bb_v, sem)
        cz.wait()
        cb.wait()

        def _keys(i, _):
            off = pl.multiple_of(i * L, L)
            bb = bb_v[pl.ds(off, L)]
            zz = zb_v[pl.ds(off, L)]
            gi = a0 + off + iota
            valid = (gi >= lo) & (gi < hi)
            key = jnp.where(valid, bb * E + zz - key_base, DUMMY)
            keys_v[pl.ds(off, L)] = key
            return 0
        lax.fori_loop(0, CPAD // L, _keys, 0)

        pltpu.sync_copy(ones_v, hist_sh.at[keys_v], add=True)
        return 0

    nchunks = (end - beg + CHUNK - 1) // CHUNK
    lax.fori_loop(0, nchunks, _chunk, 0)

    pltpu.sync_copy(hist_sh.at[pl.ds(pl.multiple_of(sid * STRIPE, 8), STRIPE)],
                    out_hbm.at[pl.ds(pl.multiple_of(seg0 * E, 8), STRIPE)])


_sc_hist = pl.kernel(
    _sc_hist_body,
    out_type=jax.ShapeDtypeStruct((B_SEG * E,), jnp.float32),
    mesh=plsc.VectorSubcoreMesh(core_axis_name="c", subcore_axis_name="s",
                                num_cores=NC, num_subcores=NS),
    scratch_types=[
        pltpu.VMEM((CPAD,), jnp.int32),
        pltpu.VMEM((CPAD,), jnp.int32),
        pltpu.VMEM((CPAD,), jnp.int32),
        pltpu.VMEM((CPAD,), jnp.float32),
        pltpu.VMEM((STRIPE,), jnp.float32),
        pltpu.VMEM((L,), jnp.int32),
        pltpu.SMEM((L,), jnp.int32),
        pltpu.VMEM_SHARED((HIST_PAD,), jnp.float32),
        pltpu.VMEM_SHARED((NS * L,), jnp.int32),
        pltpu.SemaphoreType.DMA,
    ],
)


def _mlp_body(h_ref, w1_ref, b1_ref, w2_ref, b2_ref, o_ref):
    h = h_ref[...]
    counts = jnp.sum(h, axis=1, keepdims=True)
    x = h / jnp.maximum(counts, 1.0)
    a = jnp.dot(x, w1_ref[...], preferred_element_type=jnp.float32)
    a = jnp.maximum(a + b1_ref[...][None, :], 0.0)
    o_ref[...] = (jnp.dot(a, w2_ref[...], preferred_element_type=jnp.float32)
                  + b2_ref[...][None, :])


_mlp = pl.pallas_call(
    _mlp_body,
    out_shape=jax.ShapeDtypeStruct((B_SEG, 1), jnp.float32),
)


def kernel(z, dummy1, batch, dummy2, dummy3, dummy4, W1, b1, W2, b2):
    hist = _sc_hist(z.astype(jnp.int32), batch.astype(jnp.int32))
    return _mlp(hist.reshape(B_SEG, E), W1, b1, W2, b2)


# speedup vs baseline: 1.0007x; 1.0007x over previous
import jax
import jax.numpy as jnp
from jax import lax
from jax.experimental import pallas as pl
from jax.experimental.pallas import tpu as pltpu
from jax.experimental.pallas import tpu_sc as plsc

E = 128
B_SEG = 1024
N_NODES = 100000
NC, NS, L = 2, 16, 16
SEG_PER_SC = B_SEG // NC
SEG_PER_TILE = SEG_PER_SC // NS
CHUNK = 3200
CPAD = CHUNK + 16
BINS = SEG_PER_SC * E
DUMMY = BINS
HIST_PAD = BINS + 128
STRIPE = SEG_PER_TILE * E
SAMPLE_STRIDE = 128
SAMPLES = 784


def _sc_hist_body(z_hbm, batch_hbm, out_hbm, zb_v, bb_v, keys_v, ones_v,
                  zero_v, sidx_v, samp_v, samp_s, flush_v, hist_sh, sem):
    cid = lax.axis_index("c")
    sid = lax.axis_index("s")
    seg0_sc = cid * SEG_PER_SC
    seg0 = seg0_sc + sid * SEG_PER_TILE

    def _ones(i, _):
        ones_v[pl.ds(pl.multiple_of(i * L, L), L)] = (
            jnp.full((L,), 1, jnp.int32))
        return 0
    lax.fori_loop(0, CPAD // L, _ones, 0)

    def _zero(i, _):
        zero_v[pl.ds(pl.multiple_of(i * L, L), L)] = (
            jnp.zeros((L,), jnp.int32))
        return 0
    lax.fori_loop(0, STRIPE // L, _zero, 0)
    pltpu.sync_copy(zero_v,
                    hist_sh.at[pl.ds(pl.multiple_of(sid * STRIPE, 8), STRIPE)])

    iota = lax.broadcasted_iota(jnp.int32, (L,), 0)

    def _sidx(i, _):
        base = pl.multiple_of(i * L, L)
        sidx_v[pl.ds(base, L)] = jnp.minimum(
            (base + iota) * SAMPLE_STRIDE, N_NODES - 1)
        return 0
    lax.fori_loop(0, SAMPLES // L, _sidx, 0)
    pltpu.async_copy(batch_hbm.at[sidx_v], samp_v, sem).wait()
    pslot = pl.multiple_of(HIST_PAD + sid * SAMPLES, 8)
    pltpu.sync_copy(samp_v, hist_sh.at[pl.ds(pslot, SAMPLES)])
    pltpu.sync_copy(hist_sh.at[pl.ds(pslot, SAMPLES)], samp_s)

    def _search(t):
        def _step(_, lohi):
            slo, shi = lohi
            m = (slo + shi) // 2
            v = samp_s[m]
            d = (slo < shi) & (v < t)
            u = (slo < shi) & ~(v < t)
            return (jnp.where(d, m + 1, slo), jnp.where(u, m, shi))
        r, _ = lax.fori_loop(0, 10, _step, (0, SAMPLES))
        return r

    beg = jnp.maximum(_search(seg0) - 1, 0) * SAMPLE_STRIDE
    end = jnp.minimum(_search(seg0 + SEG_PER_TILE) * SAMPLE_STRIDE, N_NODES)

    plsc.subcore_barrier()

    key_base = seg0_sc * E

    def _chunk(k, _):
        lo = beg + k * CHUNK
        hi = jnp.minimum(lo + CHUNK, end)
        a0 = pl.multiple_of(jnp.minimum(lo & ~7, N_NODES - CPAD), 8)
        cz = pltpu.async_copy(z_hbm.at[pl.ds(a0, CPAD)], zb_v, sem)
        cb = pltpu.async_copy(batch_hbm.at[pl.ds(a0, CPAD)], bb_v, sem)
        cz.wait()
        cb.wait()

        def _keys(i, _):
            off = pl.multiple_of(i * L, L)
            bb = bb_v[pl.ds(off, L)]
            zz = zb_v[pl.ds(off, L)]
            gi = a0 + off + iota
            valid = ((gi >= lo) & (gi < hi)
                     & (bb >= seg0) & (bb < seg0 + SEG_PER_TILE))
            key = jnp.where(valid, bb * E + zz - key_base, DUMMY)
            keys_v[pl.ds(off, L)] = key
            return 0
        lax.fori_loop(0, CPAD // L, _keys, 0)

        pltpu.sync_copy(ones_v, hist_sh.at[keys_v], add=True)
        return 0

    nchunks = (end - beg + CHUNK - 1) // CHUNK
    lax.fori_loop(0, nchunks, _chunk, 0)

    flush_v[pl.ds(0, L)] = jnp.full((L,), DUMMY, jnp.int32)
    pltpu.sync_copy(ones_v.at[pl.ds(0, L)], hist_sh.at[flush_v])
    plsc.subcore_barrier()

    pltpu.sync_copy(hist_sh.at[pl.ds(pl.multiple_of(sid * STRIPE, 8), STRIPE)],
                    out_hbm.at[pl.ds(pl.multiple_of(seg0 * E, 8), STRIPE)])


_sc_hist = pl.kernel(
    _sc_hist_body,
    out_type=jax.ShapeDtypeStruct((B_SEG * E,), jnp.int32),
    mesh=plsc.VectorSubcoreMesh(core_axis_name="c", subcore_axis_name="s",
                                num_cores=NC, num_subcores=NS),
    scratch_types=[
        pltpu.VMEM((CPAD,), jnp.int32),
        pltpu.VMEM((CPAD,), jnp.int32),
        pltpu.VMEM((CPAD,), jnp.int32),
        pltpu.VMEM((CPAD,), jnp.int32),
        pltpu.VMEM((STRIPE,), jnp.int32),
        pltpu.VMEM((SAMPLES,), jnp.int32),
        pltpu.VMEM((SAMPLES,), jnp.int32),
        pltpu.SMEM((SAMPLES,), jnp.int32),
        pltpu.VMEM((L,), jnp.int32),
        pltpu.VMEM_SHARED((HIST_PAD + NS * SAMPLES,), jnp.int32),
        pltpu.SemaphoreType.DMA,
    ],
)


def _mlp_body(h_ref, w1_ref, b1_ref, w2_ref, b2_ref, o_ref):
    h = h_ref[...].reshape(B_SEG, E).astype(jnp.float32)
    counts = jnp.sum(h, axis=1, keepdims=True)
    x = h / jnp.maximum(counts, 1.0)
    a = jnp.dot(x, w1_ref[...], preferred_element_type=jnp.float32)
    a = jnp.maximum(a + b1_ref[...][None, :], 0.0)
    o_ref[...] = (jnp.dot(a, w2_ref[...], preferred_element_type=jnp.float32)
                  + b2_ref[...][None, :])


_mlp = pl.pallas_call(
    _mlp_body,
    out_shape=jax.ShapeDtypeStruct((B_SEG, 1), jnp.float32),
)


def kernel(z, dummy1, batch, dummy2, dummy3, dummy4, W1, b1, W2, b2):
    hist = _sc_hist(z.astype(jnp.int32), batch.astype(jnp.int32))
    return _mlp(hist, W1, b1, W2, b2)
